# Initial kernel scaffold; baseline (speedup 1.0000x reference)
import jax
import jax.numpy as jnp
from jax.experimental import pallas as pl


def kernel(x, weight, bias):
    raise NotImplementedError("write your pallas kernel here")


# speedup vs baseline: 1.0000x; 1.0000x over previous
import jax
import jax.numpy as jnp
from jax.experimental import pallas as pl
from jax.experimental.pallas import tpu as pltpu

_LANES = 128
_TILE_BYTES = 4 << 20


def _fused_matvec_body(x_ref, w_ref, b_ref, o_ref):
    acc = jax.lax.dot_general(
        x_ref[...], w_ref[...],
        dimension_numbers=(((1,), (0,)), ((), ())),
        preferred_element_type=jnp.float32,
    )
    o_ref[...] = jnp.maximum(acc + b_ref[0, 0], 0.0).astype(o_ref.dtype)


def _row_tile(m: int, row_bytes: int) -> int:
    tile = max(8, (_TILE_BYTES // row_bytes) // 8 * 8)
    if tile >= m:
        return m
    steps = pl.cdiv(m, tile)
    if steps % 2:
        steps += 1
    return max(8, ((pl.cdiv(m, steps) + 7) // 8) * 8)


def kernel(x, weight, bias):
    n, f = x.shape
    pack = _LANES // f if (f <= _LANES and _LANES % f == 0) else 0
    bsc = bias.reshape(1, 1).astype(jnp.float32)

    if not (pack and n % pack == 0 and n // pack >= 8):
        w_row = weight.reshape(1, f).astype(jnp.float32)

        def _rowsum_body(x_ref, w_ref, b_ref, o_ref):
            prod = x_ref[...].astype(jnp.float32) * w_ref[...]
            y = jnp.sum(prod, axis=-1, keepdims=True) + b_ref[0, 0]
            o_ref[...] = jnp.maximum(y, 0.0).astype(o_ref.dtype)

        f_pad = ((f + _LANES - 1) // _LANES) * _LANES
        tn = _row_tile(n, f_pad * jnp.dtype(x.dtype).itemsize)
        return pl.pallas_call(
            _rowsum_body,
            out_shape=jax.ShapeDtypeStruct((n, 1), x.dtype),
            grid=(pl.cdiv(n, tn),),
            in_specs=[
                pl.BlockSpec((tn, f), lambda i: (i, 0)),
                pl.BlockSpec((1, f), lambda i: (0, 0)),
                pl.BlockSpec(memory_space=pltpu.MemorySpace.SMEM),
            ],
            out_specs=pl.BlockSpec((tn, 1), lambda i: (i, 0)),
            compiler_params=pltpu.CompilerParams(
                dimension_semantics=("parallel",)),
        )(x, w_row, bsc).reshape(n, 1)

    m = n // pack
    x_packed = x.reshape(m, _LANES)
    w_flat = weight.reshape(f).astype(jnp.float32)
    lane = jnp.arange(_LANES, dtype=jnp.int32)
    wd = jnp.where(
        (lane[:, None] // f) == jnp.arange(pack, dtype=jnp.int32)[None, :],
        jnp.tile(w_flat, pack)[:, None],
        0.0,
    ).astype(x.dtype)

    tm = _row_tile(m, _LANES * jnp.dtype(x.dtype).itemsize)
    out = pl.pallas_call(
        _fused_matvec_body,
        out_shape=jax.ShapeDtypeStruct((m, pack), x.dtype),
        grid=(pl.cdiv(m, tm),),
        in_specs=[
            pl.BlockSpec((tm, _LANES), lambda i: (i, 0)),
            pl.BlockSpec((_LANES, pack), lambda i: (0, 0)),
            pl.BlockSpec(memory_space=pltpu.MemorySpace.SMEM),
        ],
        out_specs=pl.BlockSpec((tm, pack), lambda i: (i, 0)),
        compiler_params=pltpu.CompilerParams(
            dimension_semantics=("parallel",)),
        cost_estimate=pl.CostEstimate(
            flops=2 * n * f, transcendentals=0,
            bytes_accessed=(n * f + n) * jnp.dtype(x.dtype).itemsize),
    )(x_packed, wd, bsc)
    return out.reshape(n, 1)
